# baseline (speedup 1.0000x reference)
import functools

import jax
import jax.numpy as jnp
from jax import lax
from jax.experimental import pallas as pl
from jax.experimental.pallas import tpu as pltpu
from jax.experimental.pallas import tpu_sc as plsc

VOCAB = 100000
D = 128
B = 16384

NUM_CORES = 2
NUM_SUBCORES = 16
LANES = 16
NW = NUM_CORES * NUM_SUBCORES
BPW = B // NW
CHUNK = 128
NCHUNK = BPW // CHUNK

_mesh = plsc.VectorSubcoreMesh(core_axis_name="c", subcore_axis_name="s")


@functools.partial(
    pl.kernel,
    mesh=_mesh,
    out_type=jax.ShapeDtypeStruct((B,), jnp.float32),
    compiler_params=pltpu.CompilerParams(needs_layout_passes=False),
    scratch_types=[
        pltpu.VMEM((CHUNK,), jnp.int32),
        pltpu.VMEM((CHUNK,), jnp.int32),
        pltpu.VMEM((CHUNK, D), jnp.float32),
        pltpu.VMEM((CHUNK, D), jnp.float32),
        pltpu.VMEM((CHUNK,), jnp.float32),
        pltpu.SemaphoreType.DMA,
        pltpu.SemaphoreType.DMA,
    ],
)
def _sc_skipgram(cw_hbm, xw_hbm, tin_hbm, tout_hbm, out_hbm,
                 ci_v, xi_v, a_v, b_v, o_v, sem_a, sem_b):
    wid = lax.axis_index("s") * NUM_CORES + lax.axis_index("c")
    base = wid * BPW
    lane = lax.iota(jnp.int32, LANES)

    for c in range(NCHUNK):
        off = base + c * CHUNK
        pltpu.sync_copy(cw_hbm.at[pl.ds(off, CHUNK)], ci_v)
        pltpu.sync_copy(xw_hbm.at[pl.ds(off, CHUNK)], xi_v)
        cp_a = pltpu.async_copy(tin_hbm.at[ci_v], a_v, sem_a)
        cp_b = pltpu.async_copy(tout_hbm.at[xi_v], b_v, sem_b)
        cp_a.wait()
        cp_b.wait()

        def group_body(g, _):
            rbase = g * LANES
            scores = jnp.zeros((LANES,), jnp.float32)
            for rl in range(LANES):
                r = rbase + rl
                acc0 = jnp.zeros((LANES,), jnp.float32)
                acc1 = jnp.zeros((LANES,), jnp.float32)
                for j in range(0, D // LANES, 2):
                    acc0 += a_v[r, pl.ds(j * LANES, LANES)] * \
                            b_v[r, pl.ds(j * LANES, LANES)]
                    acc1 += a_v[r, pl.ds((j + 1) * LANES, LANES)] * \
                            b_v[r, pl.ds((j + 1) * LANES, LANES)]
                s = jnp.sum(acc0 + acc1)
                scores = jnp.where(lane == rl, s, scores)
            o_v[pl.ds(rbase, LANES)] = scores
            return 0

        lax.fori_loop(0, CHUNK // LANES, group_body, 0)

        pltpu.sync_copy(o_v, out_hbm.at[pl.ds(off, CHUNK)])


def kernel(center_words, context_words, input_embeddings, output_embeddings):
    return _sc_skipgram(center_words.astype(jnp.int32),
                        context_words.astype(jnp.int32),
                        input_embeddings, output_embeddings)
